# baseline (speedup 1.0000x reference)
import functools

import jax
import jax.numpy as jnp
from jax import lax
from jax.experimental import pallas as pl
from jax.experimental.pallas import tpu as pltpu
from jax.experimental.pallas import tpu_sc as plsc

N = 10000
E = 320000
D = 128
ED = 16

NTILES = 32
NPT = 320
NPAD = NTILES * NPT
TRASH = NPT
ACCR = NPT + 1
SCK = 4000
NSC = E // SCK
NGRP = SCK // 16
GB = 32
CAP = SCK + GB + 16


def _prep_body(x_ref, wpre_ref, we_ref, be_ref, bpre_ref,
               a_ref, b_ref, wc_ref, cv_ref):
    xx = x_ref[...]
    wp = wpre_ref[...]
    a_ref[...] = jnp.dot(xx, wp[0:D, :], preferred_element_type=jnp.float32)
    b_ref[...] = jnp.dot(xx, wp[D:2 * D, :], preferred_element_type=jnp.float32)
    wpe = wp[2 * D:3 * D, :]
    wc_ref[...] = jnp.dot(we_ref[...], wpe, preferred_element_type=jnp.float32)
    cv_ref[...] = (jnp.dot(be_ref[...], wpe, preferred_element_type=jnp.float32)
                   + bpre_ref[...])


def _edge_body(ea_ref, wc_ref, cv_ref, c_ref):
    c_ref[...] = (jnp.dot(ea_ref[...], wc_ref[...],
                          preferred_element_type=jnp.float32) + cv_ref[...])


def _comb_body(x_ref, a_ref, s_ref, mx_ref, cnt_ref,
               wpost_ref, bpost_ref, wlin_ref, blin_ref, o_ref):
    x = x_ref[...]
    a = a_ref[...]
    s0 = s_ref[...]
    mx0 = mx_ref[...]
    cnt = cnt_ref[...]
    has = cnt > 0.0
    mean = jnp.where(has, a + s0 / jnp.maximum(cnt, 1.0), 0.0)
    mxv = jnp.where(has, a + mx0, 0.0)
    ssum = cnt * a + s0
    wp = wpost_ref[...]
    u = (jnp.dot(x, wp[0:D], preferred_element_type=jnp.float32)
         + jnp.dot(mean, wp[D:2 * D], preferred_element_type=jnp.float32)
         + jnp.dot(mxv, wp[2 * D:3 * D], preferred_element_type=jnp.float32)
         + jnp.dot(ssum, wp[3 * D:4 * D], preferred_element_type=jnp.float32)
         + bpost_ref[...])
    o_ref[...] = jnp.maximum(
        jnp.dot(u, wlin_ref[...], preferred_element_type=jnp.float32)
        + blin_ref[...], 0.0)


def _sc_body(dst_h, src_h, b_h, c_h, s_h, mx_h, cnt_h,
             dstv0, srcv0, dstv1, srcv1, eidl, srcl, locl,
             rows_b0, rows_c0, rows_b1, rows_c1,
             sumacc, maxacc, cntacc,
             sd0, ss0, sd1, ss1, sb0, sc0, sb1, sc1):
    cid = lax.axis_index("c")
    sid = lax.axis_index("s")
    wid = sid * 2 + cid
    lo = wid * NPT
    iot = lax.iota(jnp.int32, 16)
    zero16 = jnp.zeros((16,), jnp.float32)
    neg16 = jnp.full((16,), -3.0e38, jnp.float32)
    ones16 = jnp.ones((16,), jnp.float32)
    lane0 = iot == 0

    def ini(g, c):
        sumacc[pl.ds(g * 16, 16)] = zero16
        maxacc[pl.ds(g * 16, 16)] = neg16
        return c
    lax.fori_loop(0, (ACCR * D) // 16, ini, 0)

    def ini2(g, c):
        cntacc[pl.ds(g * 16, 16)] = zero16
        return c
    lax.fori_loop(0, (NPT + 16) // 16, ini2, 0)

    def stage_start(i, dv, sv, semd, sems):
        pltpu.async_copy(dst_h.at[pl.ds(i * SCK, SCK)], dv, semd)
        pltpu.async_copy(src_h.at[pl.ds(i * SCK, SCK)], sv, sems)

    def stage_wait(dv, sv, semd, sems):
        pltpu.make_async_copy(dst_h.at[pl.ds(0, SCK)], dv, semd).wait()
        pltpu.make_async_copy(src_h.at[pl.ds(0, SCK)], sv, sems).wait()

    def fire(b, rb, rc, semb, semc):
        pltpu.async_copy(b_h.at[srcl.at[pl.ds(b * GB, GB)]], rb, semb)
        pltpu.async_copy(c_h.at[eidl.at[pl.ds(b * GB, GB)]], rc, semc)

    def process(b, rb, rc, semb, semc):
        pltpu.make_async_copy(b_h.at[srcl.at[pl.ds(0, GB)]], rb, semb).wait()
        pltpu.make_async_copy(c_h.at[eidl.at[pl.ds(0, GB)]], rc, semc).wait()

        def grp(g, c):
            lgrp = locl[pl.ds(b * GB + g * 16, 16)]
            for k in range(16):
                sloc = lgrp[k]
                ab = sloc * D
                r = g * 16 + k
                vs = [rb[r, pl.ds(j * 16, 16)] + rc[r, pl.ds(j * 16, 16)]
                      for j in range(8)]
                curs = [maxacc[pl.ds(ab + j * 16, 16)] for j in range(8)]
                for j in range(8):
                    plsc.addupdate(sumacc.at[pl.ds(ab + j * 16, 16)], vs[j])
                for j in range(8):
                    maxacc[pl.ds(ab + j * 16, 16)] = jnp.maximum(curs[j], vs[j])
                lvec = jnp.full((16,), sloc, jnp.int32)
                plsc.addupdate_scatter(cntacc, [lvec], ones16, mask=lane0)
            return c
        lax.fori_loop(0, GB // 16, grp, 0)

    def flush(nfull):
        @pl.when(nfull > 0)
        def _():
            fire(0, rows_b0, rows_c0, sb0, sc0)

        def m_body(m, c):
            b0 = 2 * m
            b1 = 2 * m + 1

            @pl.when(b1 < nfull)
            def _():
                fire(b1, rows_b1, rows_c1, sb1, sc1)
            process(b0, rows_b0, rows_c0, sb0, sc0)

            @pl.when(b1 + 1 < nfull)
            def _():
                fire(b1 + 1, rows_b0, rows_c0, sb0, sc0)

            @pl.when(b1 < nfull)
            def _():
                process(b1, rows_b1, rows_c1, sb1, sc1)
            return c
        lax.fori_loop(0, (nfull + 1) // 2, m_body, 0)

    def scan_chunk(i, dv, sv, off):
        def scan(g, offv):
            d = dv[pl.ds(g * 16, 16)]
            s = sv[pl.ds(g * 16, 16)]
            m = (d >= lo) & (d < lo + NPT)
            nv = plsc.all_reduce_population_count(m)
            mi = m.astype(jnp.int32)
            pos = offv + jnp.cumsum(mi) - 1
            plsc.store_scatter(eidl, [pos], i * SCK + g * 16 + iot, mask=m)
            plsc.store_scatter(srcl, [pos], s, mask=m)
            plsc.store_scatter(locl, [pos], d - lo, mask=m)
            return offv + nv
        offv = lax.fori_loop(0, NGRP, scan, jnp.full((16,), off, jnp.int32),
                             unroll=4)
        return jnp.max(offv)

    def handle(i, dv, sv, off):
        off = scan_chunk(i, dv, sv, off)
        nfull = off // GB
        flush(nfull)
        for g in range(GB // 16):
            sl_src = pl.ds(nfull * GB + g * 16, 16)
            sl_dst = pl.ds(g * 16, 16)
            eidl[sl_dst] = eidl[sl_src]
            srcl[sl_dst] = srcl[sl_src]
            locl[sl_dst] = locl[sl_src]
        return off - nfull * GB

    stage_start(0, dstv0, srcv0, sd0, ss0)

    def super2(k, off):
        i0 = 2 * k
        i1 = 2 * k + 1
        stage_start(i1, dstv1, srcv1, sd1, ss1)
        stage_wait(dstv0, srcv0, sd0, ss0)
        off = handle(i0, dstv0, srcv0, off)

        @pl.when(i1 + 1 < NSC)
        def _():
            stage_start(i1 + 1, dstv0, srcv0, sd0, ss0)
        stage_wait(dstv1, srcv1, sd1, ss1)
        off = handle(i1, dstv1, srcv1, off)
        return off

    off = lax.fori_loop(0, NSC // 2, super2, jnp.int32(0))

    @pl.when(off > 0)
    def _():
        for g in range(GB // 16):
            p = off + g * 16 + iot
            plsc.store_scatter(locl, [p], jnp.full((16,), TRASH, jnp.int32))
            plsc.store_scatter(eidl, [p], jnp.zeros((16,), jnp.int32))
            plsc.store_scatter(srcl, [p], jnp.zeros((16,), jnp.int32))
        fire(0, rows_b0, rows_c0, sb0, sc0)
        process(0, rows_b0, rows_c0, sb0, sc0)

    pltpu.sync_copy(sumacc.at[pl.ds(0, NPT * D)], s_h.at[pl.ds(lo * D, NPT * D)])
    pltpu.sync_copy(maxacc.at[pl.ds(0, NPT * D)], mx_h.at[pl.ds(lo * D, NPT * D)])
    pltpu.sync_copy(cntacc.at[pl.ds(0, NPT)], cnt_h.at[pl.ds(lo, NPT)])


_sc_agg = functools.partial(
    pl.kernel,
    out_type=(
        jax.ShapeDtypeStruct((NPAD * D,), jnp.float32),
        jax.ShapeDtypeStruct((NPAD * D,), jnp.float32),
        jax.ShapeDtypeStruct((NPAD,), jnp.float32),
    ),
    mesh=plsc.VectorSubcoreMesh(core_axis_name="c", subcore_axis_name="s"),
    scratch_types=[
        pltpu.VMEM((SCK,), jnp.int32),
        pltpu.VMEM((SCK,), jnp.int32),
        pltpu.VMEM((SCK,), jnp.int32),
        pltpu.VMEM((SCK,), jnp.int32),
        pltpu.VMEM((CAP,), jnp.int32),
        pltpu.VMEM((CAP,), jnp.int32),
        pltpu.VMEM((CAP,), jnp.int32),
        pltpu.VMEM((GB, D), jnp.float32),
        pltpu.VMEM((GB, D), jnp.float32),
        pltpu.VMEM((GB, D), jnp.float32),
        pltpu.VMEM((GB, D), jnp.float32),
        pltpu.VMEM((ACCR * D,), jnp.float32),
        pltpu.VMEM((ACCR * D,), jnp.float32),
        pltpu.VMEM((NPT + 16,), jnp.float32),
        pltpu.SemaphoreType.DMA,
        pltpu.SemaphoreType.DMA,
        pltpu.SemaphoreType.DMA,
        pltpu.SemaphoreType.DMA,
        pltpu.SemaphoreType.DMA,
        pltpu.SemaphoreType.DMA,
        pltpu.SemaphoreType.DMA,
        pltpu.SemaphoreType.DMA,
    ],
    compiler_params=pltpu.CompilerParams(needs_layout_passes=False),
)(_sc_body)


def kernel(x, edge_index, edge_attr, We, be, Wpre, bpre, Wpost, bpost, Wlin, blin):
    src = edge_index[0]
    dst = edge_index[1]
    be2 = be.reshape(1, D)
    bpre2 = bpre.reshape(1, D)
    bpost2 = bpost.reshape(1, D)
    blin2 = blin.reshape(1, D)

    a_mat, b_mat, wc, cv = pl.pallas_call(
        _prep_body,
        out_shape=[
            jax.ShapeDtypeStruct((N, D), jnp.float32),
            jax.ShapeDtypeStruct((N, D), jnp.float32),
            jax.ShapeDtypeStruct((ED, D), jnp.float32),
            jax.ShapeDtypeStruct((1, D), jnp.float32),
        ],
    )(x, Wpre, We, be2, bpre2)

    eblk = 4000
    c_mat = pl.pallas_call(
        _edge_body,
        grid=(E // eblk,),
        in_specs=[
            pl.BlockSpec((eblk, ED), lambda i: (i, 0)),
            pl.BlockSpec((ED, D), lambda i: (0, 0)),
            pl.BlockSpec((1, D), lambda i: (0, 0)),
        ],
        out_specs=pl.BlockSpec((eblk, D), lambda i: (i, 0)),
        out_shape=jax.ShapeDtypeStruct((E, D), jnp.float32),
    )(edge_attr, wc, cv)

    s_flat, mx_flat, cnt_flat = _sc_agg(dst, src, b_mat, c_mat)
    s_mat = s_flat.reshape(NPAD, D)
    mx_mat = mx_flat.reshape(NPAD, D)
    cnt2 = cnt_flat.reshape(NPAD, 1)

    nblk = 1000
    out = pl.pallas_call(
        _comb_body,
        grid=(N // nblk,),
        in_specs=[
            pl.BlockSpec((nblk, D), lambda i: (i, 0)),
            pl.BlockSpec((nblk, D), lambda i: (i, 0)),
            pl.BlockSpec((nblk, D), lambda i: (i, 0)),
            pl.BlockSpec((nblk, D), lambda i: (i, 0)),
            pl.BlockSpec((nblk, 1), lambda i: (i, 0)),
            pl.BlockSpec((4 * D, D), lambda i: (0, 0)),
            pl.BlockSpec((1, D), lambda i: (0, 0)),
            pl.BlockSpec((D, D), lambda i: (0, 0)),
            pl.BlockSpec((1, D), lambda i: (0, 0)),
        ],
        out_specs=pl.BlockSpec((nblk, D), lambda i: (i, 0)),
        out_shape=jax.ShapeDtypeStruct((N, D), jnp.float32),
    )(x, a_mat, s_mat, mx_mat, cnt2, Wpost, bpost2, Wlin, blin2)

    return (out, edge_attr)


# speedup vs baseline: 1.5007x; 1.5007x over previous
import functools

import jax
import jax.numpy as jnp
from jax import lax
from jax.experimental import pallas as pl
from jax.experimental.pallas import tpu as pltpu
from jax.experimental.pallas import tpu_sc as plsc

N = 10000
E = 320000
D = 128
ED = 16

NTILES = 32
NPT = 320
NPAD = NTILES * NPT
TRASH = NPT
ACCR = NPT + 1

SCK = 4000
NSC = E // SCK
NGRP = SCK // 16
FCAP = 2048
SCAP = FCAP + SCK + 16
LCAP = ((E + FCAP - 1) // FCAP + 1) * FCAP

GB = 64
CH = 2048
CCAP = CH + GB + 16


def _prep_body(x_ref, wpre_ref, we_ref, be_ref, bpre_ref,
               a_ref, b_ref, wc_ref, cv_ref):
    xx = x_ref[...]
    wp = wpre_ref[...]
    a_ref[...] = jnp.dot(xx, wp[0:D, :], preferred_element_type=jnp.float32)
    b_ref[...] = jnp.dot(xx, wp[D:2 * D, :], preferred_element_type=jnp.float32)
    wpe = wp[2 * D:3 * D, :]
    wc_ref[...] = jnp.dot(we_ref[...], wpe, preferred_element_type=jnp.float32)
    cv_ref[...] = (jnp.dot(be_ref[...], wpe, preferred_element_type=jnp.float32)
                   + bpre_ref[...])


def _edge_body(ea_ref, wc_ref, cv_ref, c_ref):
    c_ref[...] = (jnp.dot(ea_ref[...], wc_ref[...],
                          preferred_element_type=jnp.float32) + cv_ref[...])


def _comb_body(x_ref, a_ref, s_ref, mx_ref, cnt_ref,
               wpost_ref, bpost_ref, wlin_ref, blin_ref, o_ref):
    x = x_ref[...]
    a = a_ref[...]
    s0 = s_ref[...]
    mx0 = mx_ref[...]
    cnt = cnt_ref[...]
    has = cnt > 0.0
    mean = jnp.where(has, a + s0 / jnp.maximum(cnt, 1.0), 0.0)
    mxv = jnp.where(has, a + mx0, 0.0)
    ssum = cnt * a + s0
    wp = wpost_ref[...]
    u = (jnp.dot(x, wp[0:D], preferred_element_type=jnp.float32)
         + jnp.dot(mean, wp[D:2 * D], preferred_element_type=jnp.float32)
         + jnp.dot(mxv, wp[2 * D:3 * D], preferred_element_type=jnp.float32)
         + jnp.dot(ssum, wp[3 * D:4 * D], preferred_element_type=jnp.float32)
         + bpost_ref[...])
    o_ref[...] = jnp.maximum(
        jnp.dot(u, wlin_ref[...], preferred_element_type=jnp.float32)
        + blin_ref[...], 0.0)


def _sc_scan_body(dst_h, src_h, eid_h, srl_h, loc_h, cnts_h,
                  dstv0, srcv0, dstv1, srcv1, eidl, srcl, locl,
                  sd0, ss0, sd1, ss1):
    cid = lax.axis_index("c")
    sid = lax.axis_index("s")
    wid = sid * 2 + cid
    lo = wid * NPT
    iot = lax.iota(jnp.int32, 16)

    def stage_start(i, dv, sv, semd, sems):
        pltpu.async_copy(dst_h.at[pl.ds(i * SCK, SCK)], dv, semd)
        pltpu.async_copy(src_h.at[pl.ds(i * SCK, SCK)], sv, sems)

    def stage_wait(dv, sv, semd, sems):
        pltpu.make_async_copy(dst_h.at[pl.ds(0, SCK)], dv, semd).wait()
        pltpu.make_async_copy(src_h.at[pl.ds(0, SCK)], sv, sems).wait()

    def scan_chunk(i, dv, sv, off):
        def scan(g, offv):
            d = dv[pl.ds(g * 16, 16)]
            s = sv[pl.ds(g * 16, 16)]
            m = (d >= lo) & (d < lo + NPT)
            nv = plsc.all_reduce_population_count(m)
            mi = m.astype(jnp.int32)
            pos = offv + jnp.cumsum(mi) - 1
            plsc.store_scatter(eidl, [pos], i * SCK + g * 16 + iot, mask=m)
            plsc.store_scatter(srcl, [pos], s, mask=m)
            plsc.store_scatter(locl, [pos], d - lo, mask=m)
            return offv + nv
        offv = lax.fori_loop(0, NGRP, scan, jnp.full((16,), off, jnp.int32),
                             unroll=2)
        return jnp.max(offv)

    def handle(i, dv, sv, carry):
        off, hbase = carry
        off = scan_chunk(i, dv, sv, off)
        nfl = off // FCAP
        hb = pl.multiple_of(hbase, FCAP)

        def fl(b, c):
            sl = pl.ds(b * FCAP, FCAP)
            hsl = pl.ds(wid * LCAP + hb + b * FCAP, FCAP)
            pltpu.sync_copy(eidl.at[sl], eid_h.at[hsl])
            pltpu.sync_copy(srcl.at[sl], srl_h.at[hsl])
            pltpu.sync_copy(locl.at[sl], loc_h.at[hsl])
            return c
        lax.fori_loop(0, nfl, fl, 0)
        rem = off - nfl * FCAP

        def sh(g, c):
            sl_src = pl.ds(nfl * FCAP + g * 16, 16)
            sl_dst = pl.ds(g * 16, 16)
            eidl[sl_dst] = eidl[sl_src]
            srcl[sl_dst] = srcl[sl_src]
            locl[sl_dst] = locl[sl_src]
            return c

        @pl.when(nfl > 0)
        def _():
            lax.fori_loop(0, (rem + 15) // 16, sh, 0)
        return (rem, hbase + nfl * FCAP)

    stage_start(0, dstv0, srcv0, sd0, ss0)

    def super2(k, carry):
        i0 = 2 * k
        i1 = 2 * k + 1
        stage_start(i1, dstv1, srcv1, sd1, ss1)
        stage_wait(dstv0, srcv0, sd0, ss0)
        carry = handle(i0, dstv0, srcv0, carry)

        @pl.when(i1 + 1 < NSC)
        def _():
            stage_start(i1 + 1, dstv0, srcv0, sd0, ss0)
        stage_wait(dstv1, srcv1, sd1, ss1)
        carry = handle(i1, dstv1, srcv1, carry)
        return carry

    off, hbase = lax.fori_loop(0, NSC // 2, super2,
                               (jnp.int32(0), jnp.int32(0)))

    @pl.when(off > 0)
    def _():
        sl = pl.ds(0, FCAP)
        hsl = pl.ds(wid * LCAP + pl.multiple_of(hbase, FCAP), FCAP)
        pltpu.sync_copy(eidl.at[sl], eid_h.at[hsl])
        pltpu.sync_copy(srcl.at[sl], srl_h.at[hsl])
        pltpu.sync_copy(locl.at[sl], loc_h.at[hsl])

    total = hbase + off
    eidl[pl.ds(0, 16)] = jnp.full((16,), total, jnp.int32)
    pltpu.sync_copy(eidl.at[pl.ds(0, 16)], cnts_h.at[pl.ds(wid * 16, 16)])


def _sc_accum_body(eid_h, srl_h, loc_h, cnts_h, b_h, c_h,
                   s_h, mx_h, cnt_h,
                   eidc0, srcc0, locc0, eidc1, srcc1, locc1,
                   rows_b0, rows_c0, rows_b1, rows_c1,
                   sumacc, maxacc, cntacc, cntv,
                   se0, se1, sb0, sc0, sb1, sc1):
    cid = lax.axis_index("c")
    sid = lax.axis_index("s")
    wid = sid * 2 + cid
    lo = wid * NPT
    iot = lax.iota(jnp.int32, 16)
    zero16 = jnp.zeros((16,), jnp.float32)
    neg16 = jnp.full((16,), -3.0e38, jnp.float32)
    ones16 = jnp.ones((16,), jnp.float32)
    lane0 = iot == 0

    def ini(g, c):
        sumacc[pl.ds(g * 16, 16)] = zero16
        maxacc[pl.ds(g * 16, 16)] = neg16
        return c
    lax.fori_loop(0, (ACCR * D) // 16, ini, 0)

    def ini2(g, c):
        cntacc[pl.ds(g * 16, 16)] = zero16
        return c
    lax.fori_loop(0, (NPT + 16) // 16, ini2, 0)

    pltpu.sync_copy(cnts_h.at[pl.ds(wid * 16, 16)], cntv)
    n = jnp.max(cntv[...])
    nch = (n + CH - 1) // CH

    def chunk_start(ci, ev, sv, lv, sem):
        sl = pl.ds(wid * LCAP + ci * CH, CH)
        pltpu.async_copy(eid_h.at[sl], ev.at[pl.ds(0, CH)], sem)
        pltpu.async_copy(srl_h.at[sl], sv.at[pl.ds(0, CH)], sem)
        pltpu.async_copy(loc_h.at[sl], lv.at[pl.ds(0, CH)], sem)

    def chunk_wait(ev, sv, lv, sem):
        pltpu.make_async_copy(eid_h.at[pl.ds(0, CH)],
                              ev.at[pl.ds(0, CH)], sem).wait()
        pltpu.make_async_copy(srl_h.at[pl.ds(0, CH)],
                              sv.at[pl.ds(0, CH)], sem).wait()
        pltpu.make_async_copy(loc_h.at[pl.ds(0, CH)],
                              lv.at[pl.ds(0, CH)], sem).wait()

    def fire(b, ev, sv, rb, rc, semb, semc):
        pltpu.async_copy(b_h.at[sv.at[pl.ds(b * GB, GB)]], rb, semb)
        pltpu.async_copy(c_h.at[ev.at[pl.ds(b * GB, GB)]], rc, semc)

    def process(b, lv, rb, rc, semb, semc):
        pltpu.make_async_copy(b_h.at[srcc0.at[pl.ds(0, GB)]], rb, semb).wait()
        pltpu.make_async_copy(c_h.at[eidc0.at[pl.ds(0, GB)]], rc, semc).wait()

        def grp(g, c):
            lgrp = lv[pl.ds(b * GB + g * 16, 16)]
            for k in range(16):
                sloc = lgrp[k]
                ab = sloc * D
                r = g * 16 + k
                vs = [rb[r, pl.ds(j * 16, 16)] + rc[r, pl.ds(j * 16, 16)]
                      for j in range(8)]
                curs = [maxacc[pl.ds(ab + j * 16, 16)] for j in range(8)]
                for j in range(8):
                    plsc.addupdate(sumacc.at[pl.ds(ab + j * 16, 16)], vs[j])
                for j in range(8):
                    maxacc[pl.ds(ab + j * 16, 16)] = jnp.maximum(curs[j], vs[j])
                lvec = jnp.full((16,), sloc, jnp.int32)
                plsc.addupdate_scatter(cntacc, [lvec], ones16, mask=lane0)
            return c
        lax.fori_loop(0, GB // 16, grp, 0)

    def do_chunk(ci, ev, sv, lv):
        nin = jnp.minimum(n - ci * CH, CH)
        nb = (nin + GB - 1) // GB

        @pl.when(nin < nb * GB)
        def _():
            for g in range(GB // 16):
                p = nin + g * 16 + iot
                plsc.store_scatter(lv, [p], jnp.full((16,), TRASH, jnp.int32))
                plsc.store_scatter(ev, [p], jnp.zeros((16,), jnp.int32))
                plsc.store_scatter(sv, [p], jnp.zeros((16,), jnp.int32))

        @pl.when(nb > 0)
        def _():
            fire(0, ev, sv, rows_b0, rows_c0, sb0, sc0)

        def m_body(m, c):
            b0 = 2 * m
            b1 = 2 * m + 1

            @pl.when(b1 < nb)
            def _():
                fire(b1, ev, sv, rows_b1, rows_c1, sb1, sc1)
            process(b0, lv, rows_b0, rows_c0, sb0, sc0)

            @pl.when(b1 + 1 < nb)
            def _():
                fire(b1 + 1, ev, sv, rows_b0, rows_c0, sb0, sc0)

            @pl.when(b1 < nb)
            def _():
                process(b1, lv, rows_b1, rows_c1, sb1, sc1)
            return c
        lax.fori_loop(0, (nb + 1) // 2, m_body, 0)

    @pl.when(nch > 0)
    def _():
        chunk_start(0, eidc0, srcc0, locc0, se0)

    def ch2(k, c):
        c0 = 2 * k
        c1 = 2 * k + 1

        @pl.when(c1 < nch)
        def _():
            chunk_start(c1, eidc1, srcc1, locc1, se1)

        chunk_wait(eidc0, srcc0, locc0, se0)
        do_chunk(c0, eidc0, srcc0, locc0)

        @pl.when(c1 + 1 < nch)
        def _():
            chunk_start(c1 + 1, eidc0, srcc0, locc0, se0)

        @pl.when(c1 < nch)
        def _():
            chunk_wait(eidc1, srcc1, locc1, se1)
            do_chunk(c1, eidc1, srcc1, locc1)
        return c
    lax.fori_loop(0, (nch + 1) // 2, ch2, 0)

    pltpu.sync_copy(sumacc.at[pl.ds(0, NPT * D)], s_h.at[pl.ds(lo * D, NPT * D)])
    pltpu.sync_copy(maxacc.at[pl.ds(0, NPT * D)], mx_h.at[pl.ds(lo * D, NPT * D)])
    pltpu.sync_copy(cntacc.at[pl.ds(0, NPT)], cnt_h.at[pl.ds(lo, NPT)])


_SC_MESH = plsc.VectorSubcoreMesh(core_axis_name="c", subcore_axis_name="s")

_sc_scan = functools.partial(
    pl.kernel,
    out_type=(
        jax.ShapeDtypeStruct((NTILES * LCAP,), jnp.int32),
        jax.ShapeDtypeStruct((NTILES * LCAP,), jnp.int32),
        jax.ShapeDtypeStruct((NTILES * LCAP,), jnp.int32),
        jax.ShapeDtypeStruct((NTILES * 16,), jnp.int32),
    ),
    mesh=_SC_MESH,
    scratch_types=[
        pltpu.VMEM((SCK,), jnp.int32),
        pltpu.VMEM((SCK,), jnp.int32),
        pltpu.VMEM((SCK,), jnp.int32),
        pltpu.VMEM((SCK,), jnp.int32),
        pltpu.VMEM((SCAP,), jnp.int32),
        pltpu.VMEM((SCAP,), jnp.int32),
        pltpu.VMEM((SCAP,), jnp.int32),
        pltpu.SemaphoreType.DMA,
        pltpu.SemaphoreType.DMA,
        pltpu.SemaphoreType.DMA,
        pltpu.SemaphoreType.DMA,
    ],
    compiler_params=pltpu.CompilerParams(needs_layout_passes=False),
)(_sc_scan_body)

_sc_accum = functools.partial(
    pl.kernel,
    out_type=(
        jax.ShapeDtypeStruct((NPAD * D,), jnp.float32),
        jax.ShapeDtypeStruct((NPAD * D,), jnp.float32),
        jax.ShapeDtypeStruct((NPAD,), jnp.float32),
    ),
    mesh=_SC_MESH,
    scratch_types=[
        pltpu.VMEM((CCAP,), jnp.int32),
        pltpu.VMEM((CCAP,), jnp.int32),
        pltpu.VMEM((CCAP,), jnp.int32),
        pltpu.VMEM((CCAP,), jnp.int32),
        pltpu.VMEM((CCAP,), jnp.int32),
        pltpu.VMEM((CCAP,), jnp.int32),
        pltpu.VMEM((GB, D), jnp.float32),
        pltpu.VMEM((GB, D), jnp.float32),
        pltpu.VMEM((GB, D), jnp.float32),
        pltpu.VMEM((GB, D), jnp.float32),
        pltpu.VMEM((ACCR * D,), jnp.float32),
        pltpu.VMEM((ACCR * D,), jnp.float32),
        pltpu.VMEM((NPT + 16,), jnp.float32),
        pltpu.VMEM((16,), jnp.int32),
        pltpu.SemaphoreType.DMA,
        pltpu.SemaphoreType.DMA,
        pltpu.SemaphoreType.DMA,
        pltpu.SemaphoreType.DMA,
        pltpu.SemaphoreType.DMA,
        pltpu.SemaphoreType.DMA,
    ],
    compiler_params=pltpu.CompilerParams(needs_layout_passes=False),
)(_sc_accum_body)


def kernel(x, edge_index, edge_attr, We, be, Wpre, bpre, Wpost, bpost, Wlin, blin):
    src = edge_index[0]
    dst = edge_index[1]
    be2 = be.reshape(1, D)
    bpre2 = bpre.reshape(1, D)
    bpost2 = bpost.reshape(1, D)
    blin2 = blin.reshape(1, D)

    eid_l, src_l, loc_l, cnts = _sc_scan(dst, src)

    a_mat, b_mat, wc, cv = pl.pallas_call(
        _prep_body,
        out_shape=[
            jax.ShapeDtypeStruct((N, D), jnp.float32),
            jax.ShapeDtypeStruct((N, D), jnp.float32),
            jax.ShapeDtypeStruct((ED, D), jnp.float32),
            jax.ShapeDtypeStruct((1, D), jnp.float32),
        ],
    )(x, Wpre, We, be2, bpre2)

    eblk = 4000
    c_mat = pl.pallas_call(
        _edge_body,
        grid=(E // eblk,),
        in_specs=[
            pl.BlockSpec((eblk, ED), lambda i: (i, 0)),
            pl.BlockSpec((ED, D), lambda i: (0, 0)),
            pl.BlockSpec((1, D), lambda i: (0, 0)),
        ],
        out_specs=pl.BlockSpec((eblk, D), lambda i: (i, 0)),
        out_shape=jax.ShapeDtypeStruct((E, D), jnp.float32),
    )(edge_attr, wc, cv)

    s_flat, mx_flat, cnt_flat = _sc_accum(eid_l, src_l, loc_l, cnts,
                                          b_mat, c_mat)
    s_mat = s_flat.reshape(NPAD, D)
    mx_mat = mx_flat.reshape(NPAD, D)
    cnt2 = cnt_flat.reshape(NPAD, 1)

    nblk = 1000
    out = pl.pallas_call(
        _comb_body,
        grid=(N // nblk,),
        in_specs=[
            pl.BlockSpec((nblk, D), lambda i: (i, 0)),
            pl.BlockSpec((nblk, D), lambda i: (i, 0)),
            pl.BlockSpec((nblk, D), lambda i: (i, 0)),
            pl.BlockSpec((nblk, D), lambda i: (i, 0)),
            pl.BlockSpec((nblk, 1), lambda i: (i, 0)),
            pl.BlockSpec((4 * D, D), lambda i: (0, 0)),
            pl.BlockSpec((1, D), lambda i: (0, 0)),
            pl.BlockSpec((D, D), lambda i: (0, 0)),
            pl.BlockSpec((1, D), lambda i: (0, 0)),
        ],
        out_specs=pl.BlockSpec((nblk, D), lambda i: (i, 0)),
        out_shape=jax.ShapeDtypeStruct((N, D), jnp.float32),
    )(x, a_mat, s_mat, mx_mat, cnt2, Wpost, bpost2, Wlin, blin2)

    return (out, edge_attr)
